# baseline (speedup 1.0000x reference)
import functools

import jax
import jax.numpy as jnp
from jax import lax
from jax.experimental import pallas as pl
from jax.experimental.pallas import tpu as pltpu
from jax.experimental.pallas import tpu_sc as plsc

AMOUNT = 0.01
S_VS_P = 0.5
_SHAPE = (64, 3, 384, 384)
_NUMEL = 64 * 3 * 384 * 384
_NIDX = int(AMOUNT * S_VS_P * _NUMEL)

_NC = 2
_NS = 16
_NW = _NC * _NS
_CHUNK = 128
_KCH = -(-_NIDX // (_NW * _CHUNK))
_PER_W = _KCH * _CHUNK
_PAD = _NW * _PER_W

_ROWS = 27648
_COLS = 1024
_BLK = 1024


def _copy_body(x_ref, o_ref):
    o_ref[...] = x_ref[...]


@jax.jit
def _tc_copy(flat2d):
    return pl.pallas_call(
        _copy_body,
        grid=(_ROWS // _BLK,),
        in_specs=[pl.BlockSpec((_BLK, _COLS), lambda i: (i, 0))],
        out_specs=pl.BlockSpec((_BLK, _COLS), lambda i: (i, 0)),
        out_shape=jax.ShapeDtypeStruct((_ROWS, _COLS), jnp.float32),
    )(flat2d)


def _sc_scatter_body(out_hbm, salt_hbm, pepper_hbm,
                     idx_s, idx_p, val1, val0, sem):
    c = lax.axis_index("c")
    s = lax.axis_index("s")
    wid = s * _NC + c
    pltpu.sync_copy(salt_hbm.at[wid], idx_s)
    pltpu.sync_copy(pepper_hbm.at[wid], idx_p)

    ones = jnp.full((16,), 1.0, jnp.float32)
    zeros = jnp.zeros((16,), jnp.float32)

    @pl.loop(0, _PER_W // 16)
    def _fill(i):
        val1[pl.ds(i * 16, 16)] = ones
        val0[pl.ds(i * 16, 16)] = zeros

    salt_dma = pltpu.make_async_copy(val1, out_hbm.at[idx_s], sem)
    pepper_dma = pltpu.make_async_copy(val0, out_hbm.at[idx_p], sem)
    salt_dma.start()
    pepper_dma.start()
    salt_dma.wait()
    pepper_dma.wait()


@functools.cache
def _sc_scatter(interpret=False):
    mesh = plsc.VectorSubcoreMesh(
        core_axis_name="c", subcore_axis_name="s",
        num_cores=_NC, num_subcores=_NS)
    return pl.kernel(
        _sc_scatter_body,
        out_type=(),
        mesh=mesh,
        interpret=interpret,
        scratch_types=[
            pltpu.VMEM((_PER_W,), jnp.int32),
            pltpu.VMEM((_PER_W,), jnp.int32),
            pltpu.VMEM((_PER_W,), jnp.float32),
            pltpu.VMEM((_PER_W,), jnp.float32),
            pltpu.SemaphoreType.DMA,
        ],
    )


def _pad_idx(idx):
    pad_n = _PAD - _NIDX
    padded = jnp.concatenate([idx, idx[:pad_n]])
    return jnp.sort(padded, stable=False).reshape(_NW, _PER_W)


def kernel(x, salt_idx, pepper_idx):
    flat = x.reshape(_NUMEL)
    salt_p = _pad_idx(salt_idx)
    pepper_p = _pad_idx(pepper_idx)
    out_ref = jax.new_ref(flat)
    _sc_scatter()(out_ref, salt_p, pepper_p)
    return out_ref[...].reshape(_SHAPE)


# speedup vs baseline: 1.0499x; 1.0499x over previous
import functools

import jax
import jax.numpy as jnp
from jax import lax
from jax.experimental import pallas as pl
from jax.experimental.pallas import tpu as pltpu
from jax.experimental.pallas import tpu_sc as plsc

AMOUNT = 0.01
S_VS_P = 0.5
_SHAPE = (64, 3, 384, 384)
_NUMEL = 64 * 3 * 384 * 384
_NIDX = int(AMOUNT * S_VS_P * _NUMEL)

_NC = 2
_NS = 16
_NW = _NC * _NS
_CHUNK = 128
_KCH = -(-_NIDX // (_NW * _CHUNK))
_PER_W = _KCH * _CHUNK
_PAD = _NW * _PER_W

_R2 = 73728
_C2 = 384
_RL = 221184
_CL = 128
_BLK = 512


def _detile_body(x_ref, o_ref):
    o_ref[...] = x_ref[...].reshape(_BLK * 3, _CL)


def _retile_body(x_ref, o_ref):
    o_ref[...] = x_ref[...].reshape(_BLK, _C2)


@jax.jit
def _tc_detile(x2d):
    return pl.pallas_call(
        _detile_body,
        grid=(_R2 // _BLK,),
        in_specs=[pl.BlockSpec((_BLK, _C2), lambda i: (i, 0))],
        out_specs=pl.BlockSpec((_BLK * 3, _CL), lambda i: (i, 0)),
        out_shape=jax.ShapeDtypeStruct((_RL, _CL), jnp.float32),
    )(x2d)


@jax.jit
def _tc_retile(lin2d):
    return pl.pallas_call(
        _retile_body,
        grid=(_R2 // _BLK,),
        in_specs=[pl.BlockSpec((_BLK * 3, _CL), lambda i: (i, 0))],
        out_specs=pl.BlockSpec((_BLK, _C2), lambda i: (i, 0)),
        out_shape=jax.ShapeDtypeStruct((_R2, _C2), jnp.float32),
    )(lin2d)


def _sc_scatter_body(out_hbm, salt_hbm, pepper_hbm,
                     idx_s, idx_p, val1, val0, sem):
    c = lax.axis_index("c")
    s = lax.axis_index("s")
    wid = s * _NC + c
    pltpu.sync_copy(salt_hbm.at[wid], idx_s)
    pltpu.sync_copy(pepper_hbm.at[wid], idx_p)

    ones = jnp.full((16,), 1.0, jnp.float32)
    zeros = jnp.zeros((16,), jnp.float32)

    @pl.loop(0, _PER_W // 16)
    def _fill(i):
        val1[pl.ds(i * 16, 16)] = ones
        val0[pl.ds(i * 16, 16)] = zeros

    salt_dma = pltpu.make_async_copy(val1, out_hbm.at[idx_s], sem)
    pepper_dma = pltpu.make_async_copy(val0, out_hbm.at[idx_p], sem)
    salt_dma.start()
    pepper_dma.start()
    salt_dma.wait()
    pepper_dma.wait()


@functools.cache
def _sc_scatter(interpret=False):
    mesh = plsc.VectorSubcoreMesh(
        core_axis_name="c", subcore_axis_name="s",
        num_cores=_NC, num_subcores=_NS)
    return pl.kernel(
        _sc_scatter_body,
        out_type=(),
        mesh=mesh,
        interpret=interpret,
        scratch_types=[
            pltpu.VMEM((_PER_W,), jnp.int32),
            pltpu.VMEM((_PER_W,), jnp.int32),
            pltpu.VMEM((_PER_W,), jnp.float32),
            pltpu.VMEM((_PER_W,), jnp.float32),
            pltpu.SemaphoreType.DMA,
        ],
    )


def _pad_idx(idx):
    pad_n = _PAD - _NIDX
    return jnp.concatenate([idx, idx[:pad_n]]).reshape(_NW, _PER_W)


def kernel(x, salt_idx, pepper_idx):
    flat = _tc_detile(x.reshape(_R2, _C2)).reshape(_NUMEL)
    salt_p = _pad_idx(salt_idx)
    pepper_p = _pad_idx(pepper_idx)
    out_ref = jax.new_ref(flat)
    _sc_scatter()(out_ref, salt_p, pepper_p)
    return _tc_retile(out_ref[...].reshape(_RL, _CL)).reshape(_SHAPE)


# speedup vs baseline: 1.2964x; 1.2348x over previous
import functools

import jax
import jax.numpy as jnp
from jax import lax
from jax.experimental import pallas as pl
from jax.experimental.pallas import tpu as pltpu
from jax.experimental.pallas import tpu_sc as plsc

AMOUNT = 0.01
S_VS_P = 0.5
_SHAPE = (64, 3, 384, 384)
_NUMEL = 64 * 3 * 384 * 384
_NIDX = int(AMOUNT * S_VS_P * _NUMEL)

_NC = 2
_NS = 16
_NW = _NC * _NS
_CHUNK = 128
_KCH = -(-_NIDX // (_NW * _CHUNK))
_PER_W = _KCH * _CHUNK
_PAD = _NW * _PER_W

_R2 = 73728
_C2 = 384
_RL = 221184
_CL = 128
_BLK = 2048


def _detile_body(x_ref, o_ref):
    o_ref[...] = x_ref[...].reshape(_BLK * 3, _CL)


def _retile_body(x_ref, o_ref):
    o_ref[...] = x_ref[...].reshape(_BLK, _C2)


@jax.jit
def _tc_detile(x2d):
    return pl.pallas_call(
        _detile_body,
        grid=(_R2 // _BLK,),
        in_specs=[pl.BlockSpec((_BLK, _C2), lambda i: (i, 0))],
        out_specs=pl.BlockSpec((_BLK * 3, _CL), lambda i: (i, 0)),
        out_shape=jax.ShapeDtypeStruct((_RL, _CL), jnp.float32),
    )(x2d)


@jax.jit
def _tc_retile(lin2d):
    return pl.pallas_call(
        _retile_body,
        grid=(_R2 // _BLK,),
        in_specs=[pl.BlockSpec((_BLK * 3, _CL), lambda i: (i, 0))],
        out_specs=pl.BlockSpec((_BLK, _C2), lambda i: (i, 0)),
        out_shape=jax.ShapeDtypeStruct((_R2, _C2), jnp.float32),
    )(lin2d)


def _sc_scatter_body(out_hbm, salt_hbm, pepper_hbm,
                     idx_s, idx_p, val1, val0, sem):
    c = lax.axis_index("c")
    s = lax.axis_index("s")
    wid = s * _NC + c
    pltpu.sync_copy(salt_hbm.at[wid], idx_s)
    pltpu.sync_copy(pepper_hbm.at[wid], idx_p)

    ones = jnp.full((16,), 1.0, jnp.float32)
    zeros = jnp.zeros((16,), jnp.float32)

    @pl.loop(0, _PER_W // 16)
    def _fill(i):
        val1[pl.ds(i * 16, 16)] = ones
        val0[pl.ds(i * 16, 16)] = zeros

    salt_dma = pltpu.make_async_copy(val1, out_hbm.at[idx_s], sem)
    pepper_dma = pltpu.make_async_copy(val0, out_hbm.at[idx_p], sem)
    salt_dma.start()
    pepper_dma.start()
    salt_dma.wait()
    pepper_dma.wait()


@functools.cache
def _sc_scatter(interpret=False):
    mesh = plsc.VectorSubcoreMesh(
        core_axis_name="c", subcore_axis_name="s",
        num_cores=_NC, num_subcores=_NS)
    return pl.kernel(
        _sc_scatter_body,
        out_type=(),
        mesh=mesh,
        interpret=interpret,
        scratch_types=[
            pltpu.VMEM((_PER_W,), jnp.int32),
            pltpu.VMEM((_PER_W,), jnp.int32),
            pltpu.VMEM((_PER_W,), jnp.float32),
            pltpu.VMEM((_PER_W,), jnp.float32),
            pltpu.SemaphoreType.DMA,
        ],
    )


def _pad_idx(idx):
    pad_n = _PAD - _NIDX
    return jnp.concatenate([idx, idx[:pad_n]]).reshape(_NW, _PER_W)


def kernel(x, salt_idx, pepper_idx):
    flat = _tc_detile(x.reshape(_R2, _C2)).reshape(_NUMEL)
    salt_p = _pad_idx(salt_idx)
    pepper_p = _pad_idx(pepper_idx)
    out_ref = jax.new_ref(flat)
    _sc_scatter()(out_ref, salt_p, pepper_p)
    return _tc_retile(out_ref[...].reshape(_RL, _CL)).reshape(_SHAPE)


# speedup vs baseline: 1.3397x; 1.0334x over previous
import functools

import jax
import jax.numpy as jnp
from jax import lax
from jax.experimental import pallas as pl
from jax.experimental.pallas import tpu as pltpu
from jax.experimental.pallas import tpu_sc as plsc

AMOUNT = 0.01
S_VS_P = 0.5
_SHAPE = (64, 3, 384, 384)
_NUMEL = 64 * 3 * 384 * 384
_NIDX = int(AMOUNT * S_VS_P * _NUMEL)

_NC = 2
_NS = 16
_NW = _NC * _NS
_CHUNK = 128
_KCH = -(-_NIDX // (_NW * _CHUNK))
_PER_W = _KCH * _CHUNK
_PAD = _NW * _PER_W

_R2 = 73728
_C2 = 384
_RL = 221184
_CL = 128
_BLK = 4608


def _detile_body(x_ref, o_ref):
    o_ref[...] = x_ref[...].reshape(_BLK * 3, _CL)


def _retile_body(x_ref, o_ref):
    o_ref[...] = x_ref[...].reshape(_BLK, _C2)


@jax.jit
def _tc_detile(x2d):
    return pl.pallas_call(
        _detile_body,
        grid=(_R2 // _BLK,),
        in_specs=[pl.BlockSpec((_BLK, _C2), lambda i: (i, 0))],
        out_specs=pl.BlockSpec((_BLK * 3, _CL), lambda i: (i, 0)),
        out_shape=jax.ShapeDtypeStruct((_RL, _CL), jnp.float32),
    )(x2d)


@jax.jit
def _tc_retile(lin2d):
    return pl.pallas_call(
        _retile_body,
        grid=(_R2 // _BLK,),
        in_specs=[pl.BlockSpec((_BLK * 3, _CL), lambda i: (i, 0))],
        out_specs=pl.BlockSpec((_BLK, _C2), lambda i: (i, 0)),
        out_shape=jax.ShapeDtypeStruct((_R2, _C2), jnp.float32),
    )(lin2d)


def _sc_scatter_body(out_hbm, salt_hbm, pepper_hbm,
                     idx_s, idx_p, val1, val0, sem):
    c = lax.axis_index("c")
    s = lax.axis_index("s")
    wid = s * _NC + c
    pltpu.sync_copy(salt_hbm.at[wid], idx_s)
    pltpu.sync_copy(pepper_hbm.at[wid], idx_p)

    ones = jnp.full((16,), 1.0, jnp.float32)
    zeros = jnp.zeros((16,), jnp.float32)

    @pl.loop(0, _PER_W // 16)
    def _fill(i):
        val1[pl.ds(i * 16, 16)] = ones
        val0[pl.ds(i * 16, 16)] = zeros

    salt_dma = pltpu.make_async_copy(val1, out_hbm.at[idx_s], sem)
    pepper_dma = pltpu.make_async_copy(val0, out_hbm.at[idx_p], sem)
    salt_dma.start()
    pepper_dma.start()
    salt_dma.wait()
    pepper_dma.wait()


@functools.cache
def _sc_scatter(interpret=False):
    mesh = plsc.VectorSubcoreMesh(
        core_axis_name="c", subcore_axis_name="s",
        num_cores=_NC, num_subcores=_NS)
    return pl.kernel(
        _sc_scatter_body,
        out_type=(),
        mesh=mesh,
        interpret=interpret,
        scratch_types=[
            pltpu.VMEM((_PER_W,), jnp.int32),
            pltpu.VMEM((_PER_W,), jnp.int32),
            pltpu.VMEM((_PER_W,), jnp.float32),
            pltpu.VMEM((_PER_W,), jnp.float32),
            pltpu.SemaphoreType.DMA,
        ],
    )


def _pad_idx(idx):
    pad_n = _PAD - _NIDX
    return jnp.concatenate([idx, idx[:pad_n]]).reshape(_NW, _PER_W)


def kernel(x, salt_idx, pepper_idx):
    flat = _tc_detile(x.reshape(_R2, _C2)).reshape(_NUMEL)
    salt_p = _pad_idx(salt_idx)
    pepper_p = _pad_idx(pepper_idx)
    out_ref = jax.new_ref(flat)
    _sc_scatter()(out_ref, salt_p, pepper_p)
    return _tc_retile(out_ref[...].reshape(_RL, _CL)).reshape(_SHAPE)
